# baseline (speedup 1.0000x reference)
import functools

import jax
import jax.numpy as jnp
from jax import lax
from jax.experimental import pallas as pl
from jax.experimental.pallas import tpu as pltpu
from jax.experimental.pallas import tpu_sc as plsc

_N_REL = 1000
_DE = 128
_DR = 64
_B = 4096

_NC = 2
_NS = 16
_NW = _NC * _NS
_BPW = _B // _NW
_NGRP = _BPW // 16

_RBLK = 40


def _s_reduce_body(mr_ref, rel_ref, s_ref):
    x = mr_ref[...]
    acc = x[:, 0:_DE]
    for k in range(1, _DR):
        acc = acc + x[:, k * _DE:(k + 1) * _DE]
    s_ref[:, 0:_DE] = acc
    r = rel_ref[...]
    s_ref[:, _DE:_DE + _DR] = r
    s_ref[:, _DE + _DR:] = jnp.zeros((_RBLK, _DE - _DR), jnp.float32)


def _compute_s(mr, relations):
    return pl.pallas_call(
        _s_reduce_body,
        grid=(_N_REL // _RBLK,),
        in_specs=[pl.BlockSpec((_RBLK, _DE * _DR), lambda i: (i, 0)),
                  pl.BlockSpec((_RBLK, _DR), lambda i: (i, 0))],
        out_specs=pl.BlockSpec((_RBLK, 2 * _DE), lambda i: (i, 0)),
        out_shape=jax.ShapeDtypeStruct((_N_REL, 2 * _DE), jnp.float32),
    )(mr, relations)


def _nr_rsqrt(x):
    xi = plsc.bitcast(x, jnp.int32)
    yi = jnp.full((16,), 0x5F3759DF, jnp.int32) - (xi >> 1)
    y = plsc.bitcast(yi, jnp.float32)
    for _ in range(3):
        y = y * (1.5 - 0.5 * x * y * y)
    return jnp.minimum(y, 1e12)


_UNROLL = 4


def _sc_body(ih_hbm, it_hbm, ir_hbm, ich_hbm, ict_hbm, ent_hbm,
             s_hbm, dpos_hbm, dneg_hbm,
             ihv, itv, irv, ichv, ictv, hv, tv, chv, ctv, sv,
             dposv, dnegv, sem):
    wid = lax.axis_index("s") * _NC + lax.axis_index("c")
    base = wid * _BPW

    pltpu.sync_copy(ih_hbm.at[pl.ds(base, _BPW)], ihv)
    pltpu.sync_copy(it_hbm.at[pl.ds(base, _BPW)], itv)
    pltpu.sync_copy(ir_hbm.at[pl.ds(base, _BPW)], irv)
    pltpu.sync_copy(ich_hbm.at[pl.ds(base, _BPW)], ichv)
    pltpu.sync_copy(ict_hbm.at[pl.ds(base, _BPW)], ictv)

    cps = [
        pltpu.async_copy(ent_hbm.at[ihv], hv, sem),
        pltpu.async_copy(ent_hbm.at[itv], tv, sem),
        pltpu.async_copy(ent_hbm.at[ichv], chv, sem),
        pltpu.async_copy(ent_hbm.at[ictv], ctv, sem),
        pltpu.async_copy(s_hbm.at[irv], sv, sem),
    ]
    for c in cps:
        c.wait()

    zero = jnp.zeros((16,), jnp.float32)
    for g in range(_NGRP):
        rows = g * 16 + lax.iota(jnp.int32, 16)

        def body(jj, acc):
            for u in range(_UNROLL):
                (nh2, nt2, hrr, hrt, rtr, nch2, nct2, crr, crct, rct, rr) = acc
                cols = jnp.full((16,), jj * _UNROLL + u, jnp.int32)
                cols64 = cols + _DR
                sj = plsc.load_gather(sv, [rows, cols])
                sj64 = plsc.load_gather(sv, [rows, cols64])
                hj = plsc.load_gather(hv, [rows, cols])
                hj64 = plsc.load_gather(hv, [rows, cols64])
                tj = plsc.load_gather(tv, [rows, cols])
                tj64 = plsc.load_gather(tv, [rows, cols64])
                chj = plsc.load_gather(chv, [rows, cols])
                chj64 = plsc.load_gather(chv, [rows, cols64])
                ctj = plsc.load_gather(ctv, [rows, cols])
                ctj64 = plsc.load_gather(ctv, [rows, cols64])
                rj = plsc.load_gather(sv, [rows, cols + 2 * _DR])
                hr = hj * sj + hj64 * sj64
                tr = tj * sj + tj64 * sj64
                cr = chj * sj + chj64 * sj64
                ctr = ctj * sj + ctj64 * sj64
                acc = (nh2 + hr * hr, nt2 + tr * tr, hrr + hr * rj,
                       hrt + hr * tr, rtr + rj * tr,
                       nch2 + cr * cr, nct2 + ctr * ctr, crr + cr * rj,
                       crct + cr * ctr, rct + rj * ctr, rr + rj * rj)
            return acc

        acc0 = (zero,) * 11
        (nh2, nt2, hrr, hrt, rtr, nch2, nct2, crr, crct, rct, rr) = \
            lax.fori_loop(0, _DR // _UNROLL, body, acc0)

        ivh = _nr_rsqrt(nh2)
        ivt = _nr_rsqrt(nt2)
        ivch = _nr_rsqrt(nch2)
        ivct = _nr_rsqrt(nct2)
        d_pos = (nh2 * ivh * ivh + rr + nt2 * ivt * ivt
                 + 2.0 * (hrr * ivh - hrt * ivh * ivt - rtr * ivt))
        d_neg = (nch2 * ivch * ivch + rr + nct2 * ivct * ivct
                 + 2.0 * (crr * ivch - crct * ivch * ivct - rct * ivct))
        dposv[pl.ds(g * 16, 16)] = d_pos
        dnegv[pl.ds(g * 16, 16)] = d_neg

    pltpu.sync_copy(dposv, dpos_hbm.at[pl.ds(base, _BPW)])
    pltpu.sync_copy(dnegv, dneg_hbm.at[pl.ds(base, _BPW)])


_sc_call = functools.partial(
    pl.kernel,
    out_type=[jax.ShapeDtypeStruct((_B,), jnp.float32),
              jax.ShapeDtypeStruct((_B,), jnp.float32)],
    mesh=plsc.VectorSubcoreMesh(core_axis_name="c", subcore_axis_name="s",
                                num_cores=_NC, num_subcores=_NS),
    scratch_types=[
        pltpu.VMEM((_BPW,), jnp.int32),
        pltpu.VMEM((_BPW,), jnp.int32),
        pltpu.VMEM((_BPW,), jnp.int32),
        pltpu.VMEM((_BPW,), jnp.int32),
        pltpu.VMEM((_BPW,), jnp.int32),
        pltpu.VMEM((_BPW, _DE), jnp.float32),
        pltpu.VMEM((_BPW, _DE), jnp.float32),
        pltpu.VMEM((_BPW, _DE), jnp.float32),
        pltpu.VMEM((_BPW, _DE), jnp.float32),
        pltpu.VMEM((_BPW, 2 * _DE), jnp.float32),
        pltpu.VMEM((_BPW,), jnp.float32),
        pltpu.VMEM((_BPW,), jnp.float32),
        pltpu.SemaphoreType.DMA,
    ],
    compiler_params=pltpu.CompilerParams(needs_layout_passes=False),
)(_sc_body)


def kernel(data, entities, relations, mr):
    s = _compute_s(mr, relations)
    ih = data[:, 0]
    it = data[:, 1]
    ir = data[:, 2]
    ich = data[:, 3]
    ict = data[:, 4]
    d_pos, d_neg = _sc_call(ih, it, ir, ich, ict, entities, s)
    t_lab = -jnp.ones((_B, 1), dtype=jnp.float32)
    return (d_pos, d_neg, t_lab)


# speedup vs baseline: 1.0024x; 1.0024x over previous
import functools

import jax
import jax.numpy as jnp
from jax import lax
from jax.experimental import pallas as pl
from jax.experimental.pallas import tpu as pltpu
from jax.experimental.pallas import tpu_sc as plsc

_N_REL = 1000
_DE = 128
_DR = 64
_B = 4096

_NC = 2
_NS = 16
_NW = _NC * _NS
_BPW = _B // _NW
_NGRP = _BPW // 16

_RBLK = 40


def _s_reduce_body(mr_ref, rel_ref, s_ref):
    x = mr_ref[...]
    acc = x[:, 0:_DE]
    for k in range(1, _DR):
        acc = acc + x[:, k * _DE:(k + 1) * _DE]
    s_ref[:, 0:_DE] = acc
    r = rel_ref[...]
    s_ref[:, _DE:_DE + _DR] = r
    s_ref[:, _DE + _DR:] = jnp.zeros((_RBLK, _DE - _DR), jnp.float32)


def _compute_s(mr, relations):
    return pl.pallas_call(
        _s_reduce_body,
        grid=(_N_REL // _RBLK,),
        in_specs=[pl.BlockSpec((_RBLK, _DE * _DR), lambda i: (i, 0)),
                  pl.BlockSpec((_RBLK, _DR), lambda i: (i, 0))],
        out_specs=pl.BlockSpec((_RBLK, 2 * _DE), lambda i: (i, 0)),
        out_shape=jax.ShapeDtypeStruct((_N_REL, 2 * _DE), jnp.float32),
    )(mr, relations)


def _nr_rsqrt(x):
    xi = plsc.bitcast(x, jnp.int32)
    yi = jnp.full((16,), 0x5F3759DF, jnp.int32) - (xi >> 1)
    y = plsc.bitcast(yi, jnp.float32)
    for _ in range(3):
        y = y * (1.5 - 0.5 * x * y * y)
    return jnp.minimum(y, 1e12)


_CHUNK = 32
_NCHUNK = _BPW // _CHUNK
_GPC = _CHUNK // 16


def _sc_body(ih_hbm, it_hbm, ir_hbm, ich_hbm, ict_hbm, ent_hbm,
             s_hbm, dpos_hbm, dneg_hbm,
             ihv, itv, irv, ichv, ictv, hv, tv, chv, ctv, sv,
             dposv, dnegv, sem0, sem1, sem2, sem3):
    wid = lax.axis_index("s") * _NC + lax.axis_index("c")
    base = wid * _BPW

    pltpu.sync_copy(ih_hbm.at[pl.ds(base, _BPW)], ihv)
    pltpu.sync_copy(it_hbm.at[pl.ds(base, _BPW)], itv)
    pltpu.sync_copy(ir_hbm.at[pl.ds(base, _BPW)], irv)
    pltpu.sync_copy(ich_hbm.at[pl.ds(base, _BPW)], ichv)
    pltpu.sync_copy(ict_hbm.at[pl.ds(base, _BPW)], ictv)

    sems = [sem0, sem1, sem2, sem3]
    descs = []
    for k in range(_NCHUNK):
        sl = pl.ds(k * _CHUNK, _CHUNK)
        descs.append([
            pltpu.async_copy(ent_hbm.at[ihv.at[sl]], hv.at[sl], sems[k]),
            pltpu.async_copy(ent_hbm.at[itv.at[sl]], tv.at[sl], sems[k]),
            pltpu.async_copy(ent_hbm.at[ichv.at[sl]], chv.at[sl], sems[k]),
            pltpu.async_copy(ent_hbm.at[ictv.at[sl]], ctv.at[sl], sems[k]),
            pltpu.async_copy(s_hbm.at[irv.at[sl]], sv.at[sl], sems[k]),
        ])

    zero = jnp.zeros((16,), jnp.float32)
    for k in range(_NCHUNK):
        for d in descs[k]:
            d.wait()
        for g in range(k * _GPC, (k + 1) * _GPC):
            rows = g * 16 + lax.iota(jnp.int32, 16)

            def body(j, acc):
                (nh2, nt2, hrr, hrt, rtr, nch2, nct2, crr, crct, rct, rr) = acc
                cols = jnp.full((16,), j, jnp.int32)
                cols64 = cols + _DR
                sj = plsc.load_gather(sv, [rows, cols])
                sj64 = plsc.load_gather(sv, [rows, cols64])
                hj = plsc.load_gather(hv, [rows, cols])
                hj64 = plsc.load_gather(hv, [rows, cols64])
                tj = plsc.load_gather(tv, [rows, cols])
                tj64 = plsc.load_gather(tv, [rows, cols64])
                chj = plsc.load_gather(chv, [rows, cols])
                chj64 = plsc.load_gather(chv, [rows, cols64])
                ctj = plsc.load_gather(ctv, [rows, cols])
                ctj64 = plsc.load_gather(ctv, [rows, cols64])
                rj = plsc.load_gather(sv, [rows, cols + 2 * _DR])
                hr = hj * sj + hj64 * sj64
                tr = tj * sj + tj64 * sj64
                cr = chj * sj + chj64 * sj64
                ctr = ctj * sj + ctj64 * sj64
                return (nh2 + hr * hr, nt2 + tr * tr, hrr + hr * rj,
                        hrt + hr * tr, rtr + rj * tr,
                        nch2 + cr * cr, nct2 + ctr * ctr, crr + cr * rj,
                        crct + cr * ctr, rct + rj * ctr, rr + rj * rj)

            acc0 = (zero,) * 11
            (nh2, nt2, hrr, hrt, rtr, nch2, nct2, crr, crct, rct, rr) = \
                plsc.parallel_loop(0, _DR, unroll=4, carry=acc0)(body)

            ivh = _nr_rsqrt(nh2)
            ivt = _nr_rsqrt(nt2)
            ivch = _nr_rsqrt(nch2)
            ivct = _nr_rsqrt(nct2)
            d_pos = (nh2 * ivh * ivh + rr + nt2 * ivt * ivt
                     + 2.0 * (hrr * ivh - hrt * ivh * ivt - rtr * ivt))
            d_neg = (nch2 * ivch * ivch + rr + nct2 * ivct * ivct
                     + 2.0 * (crr * ivch - crct * ivch * ivct - rct * ivct))
            dposv[pl.ds(g * 16, 16)] = d_pos
            dnegv[pl.ds(g * 16, 16)] = d_neg

    pltpu.sync_copy(dposv, dpos_hbm.at[pl.ds(base, _BPW)])
    pltpu.sync_copy(dnegv, dneg_hbm.at[pl.ds(base, _BPW)])


_sc_call = functools.partial(
    pl.kernel,
    out_type=[jax.ShapeDtypeStruct((_B,), jnp.float32),
              jax.ShapeDtypeStruct((_B,), jnp.float32)],
    mesh=plsc.VectorSubcoreMesh(core_axis_name="c", subcore_axis_name="s",
                                num_cores=_NC, num_subcores=_NS),
    scratch_types=[
        pltpu.VMEM((_BPW,), jnp.int32),
        pltpu.VMEM((_BPW,), jnp.int32),
        pltpu.VMEM((_BPW,), jnp.int32),
        pltpu.VMEM((_BPW,), jnp.int32),
        pltpu.VMEM((_BPW,), jnp.int32),
        pltpu.VMEM((_BPW, _DE), jnp.float32),
        pltpu.VMEM((_BPW, _DE), jnp.float32),
        pltpu.VMEM((_BPW, _DE), jnp.float32),
        pltpu.VMEM((_BPW, _DE), jnp.float32),
        pltpu.VMEM((_BPW, 2 * _DE), jnp.float32),
        pltpu.VMEM((_BPW,), jnp.float32),
        pltpu.VMEM((_BPW,), jnp.float32),
        pltpu.SemaphoreType.DMA,
        pltpu.SemaphoreType.DMA,
        pltpu.SemaphoreType.DMA,
        pltpu.SemaphoreType.DMA,
    ],
    compiler_params=pltpu.CompilerParams(needs_layout_passes=False),
)(_sc_body)


def kernel(data, entities, relations, mr):
    s = _compute_s(mr, relations)
    ih = data[:, 0]
    it = data[:, 1]
    ir = data[:, 2]
    ich = data[:, 3]
    ict = data[:, 4]
    d_pos, d_neg = _sc_call(ih, it, ir, ich, ict, entities, s)
    t_lab = -jnp.ones((_B, 1), dtype=jnp.float32)
    return (d_pos, d_neg, t_lab)


# speedup vs baseline: 1.6910x; 1.6870x over previous
import functools

import jax
import jax.numpy as jnp
from jax import lax
from jax.experimental import pallas as pl
from jax.experimental.pallas import tpu as pltpu
from jax.experimental.pallas import tpu_sc as plsc

_N_REL = 1000
_DE = 128
_DR = 64
_B = 4096

_NC = 2
_NS = 16
_NW = _NC * _NS
_BPW = _B // _NW
_NGRP = _BPW // 16

_RBLK = 40


def _s_reduce_body(mr_ref, rel_ref, s_ref):
    x = mr_ref[...]
    acc = x[:, 0:_DE]
    for k in range(1, _DR):
        acc = acc + x[:, k * _DE:(k + 1) * _DE]
    s_ref[:, 0:_DE] = acc
    r = rel_ref[...]
    s_ref[:, _DE:_DE + _DR] = r
    s_ref[:, _DE + _DR:] = jnp.zeros((_RBLK, _DE - _DR), jnp.float32)


def _compute_s(mr, relations):
    return pl.pallas_call(
        _s_reduce_body,
        grid=(_N_REL // _RBLK,),
        in_specs=[pl.BlockSpec((_RBLK, _DE * _DR), lambda i: (i, 0)),
                  pl.BlockSpec((_RBLK, _DR), lambda i: (i, 0))],
        out_specs=pl.BlockSpec((_RBLK, 2 * _DE), lambda i: (i, 0)),
        out_shape=jax.ShapeDtypeStruct((_N_REL, 2 * _DE), jnp.float32),
    )(mr, relations)


def _nr_rsqrt(x):
    xi = plsc.bitcast(x, jnp.int32)
    yi = jnp.full((16,), 0x5F3759DF, jnp.int32) - (xi >> 1)
    y = plsc.bitcast(yi, jnp.float32)
    for _ in range(3):
        y = y * (1.5 - 0.5 * x * y * y)
    return jnp.minimum(y, 1e12)


_CHUNK = 32
_NCHUNK = _BPW // _CHUNK
_GPC = _CHUNK // 16


def _sc_body(ih_hbm, it_hbm, ir_hbm, ich_hbm, ict_hbm, ent_hbm,
             s_hbm, dpos_hbm, dneg_hbm,
             ihv, itv, irv, ichv, ictv, hv, tv, chv, ctv, sv,
             dposv, dnegv, sem0, sem1, sem2, sem3):
    sid = lax.axis_index("s")
    wid = sid * _NC + lax.axis_index("c")
    base = wid * _BPW

    pltpu.sync_copy(ih_hbm.at[pl.ds(base, _BPW)], ihv)
    pltpu.sync_copy(it_hbm.at[pl.ds(base, _BPW)], itv)
    pltpu.sync_copy(ir_hbm.at[pl.ds(base, _BPW)], irv)
    pltpu.sync_copy(ich_hbm.at[pl.ds(base, _BPW)], ichv)
    pltpu.sync_copy(ict_hbm.at[pl.ds(base, _BPW)], ictv)

    sems = [sem0, sem1, sem2, sem3]
    descs = []
    for k in range(_NCHUNK):
        sl = pl.ds(k * _CHUNK, _CHUNK)
        descs.append([
            pltpu.async_copy(ent_hbm.at[ihv.at[sl]], hv.at[sl], sems[k]),
            pltpu.async_copy(ent_hbm.at[itv.at[sl]], tv.at[sl], sems[k]),
            pltpu.async_copy(ent_hbm.at[ichv.at[sl]], chv.at[sl], sems[k]),
            pltpu.async_copy(ent_hbm.at[ictv.at[sl]], ctv.at[sl], sems[k]),
            pltpu.async_copy(s_hbm.at[irv.at[sl]], sv.at[sl], sems[k]),
        ])

    def _dot4(a, b):
        acc = a[0] * b[0]
        for q in range(1, 4):
            acc = acc + a[q] * b[q]
        return plsc.cumsum(acc)

    lane15 = lax.iota(jnp.int32, 16) == 15

    for k in range(_NCHUNK):
        for d in descs[k]:
            d.wait()

        def sbody(i):
            hq = [hv[i, pl.ds(q * 16, 16)] for q in range(8)]
            tq = [tv[i, pl.ds(q * 16, 16)] for q in range(8)]
            chq = [chv[i, pl.ds(q * 16, 16)] for q in range(8)]
            ctq = [ctv[i, pl.ds(q * 16, 16)] for q in range(8)]
            sq = [sv[i, pl.ds(q * 16, 16)] for q in range(12)]
            hr = [hq[q] * sq[q] + hq[q + 4] * sq[q + 4] for q in range(4)]
            tr = [tq[q] * sq[q] + tq[q + 4] * sq[q + 4] for q in range(4)]
            cr = [chq[q] * sq[q] + chq[q + 4] * sq[q + 4] for q in range(4)]
            ctr = [ctq[q] * sq[q] + ctq[q + 4] * sq[q + 4] for q in range(4)]
            rl = sq[8:12]
            nh2 = _dot4(hr, hr)
            nt2 = _dot4(tr, tr)
            hrr = _dot4(hr, rl)
            hrt = _dot4(hr, tr)
            rtr = _dot4(rl, tr)
            nch2 = _dot4(cr, cr)
            nct2 = _dot4(ctr, ctr)
            crr = _dot4(cr, rl)
            crct = _dot4(cr, ctr)
            rct = _dot4(rl, ctr)
            rr = _dot4(rl, rl)
            ivh = _nr_rsqrt(nh2)
            ivt = _nr_rsqrt(nt2)
            ivch = _nr_rsqrt(nch2)
            ivct = _nr_rsqrt(nct2)
            d_pos = (nh2 * ivh * ivh + rr + nt2 * ivt * ivt
                     + 2.0 * (hrr * ivh - hrt * ivh * ivt - rtr * ivt))
            d_neg = (nch2 * ivch * ivch + rr + nct2 * ivct * ivct
                     + 2.0 * (crr * ivch - crct * ivch * ivct - rct * ivct))
            idxi = jnp.full((16,), i, jnp.int32)
            plsc.store_scatter(dposv, [idxi], d_pos, mask=lane15)
            plsc.store_scatter(dnegv, [idxi], d_neg, mask=lane15)

        plsc.parallel_loop(k * _CHUNK, (k + 1) * _CHUNK, unroll=2)(sbody)

    pltpu.sync_copy(dposv, dpos_hbm.at[pl.ds(base, _BPW)])
    pltpu.sync_copy(dnegv, dneg_hbm.at[pl.ds(base, _BPW)])


_sc_call = functools.partial(
    pl.kernel,
    out_type=[jax.ShapeDtypeStruct((_B,), jnp.float32),
              jax.ShapeDtypeStruct((_B,), jnp.float32)],
    mesh=plsc.VectorSubcoreMesh(core_axis_name="c", subcore_axis_name="s",
                                num_cores=_NC, num_subcores=_NS),
    scratch_types=[
        pltpu.VMEM((_BPW,), jnp.int32),
        pltpu.VMEM((_BPW,), jnp.int32),
        pltpu.VMEM((_BPW,), jnp.int32),
        pltpu.VMEM((_BPW,), jnp.int32),
        pltpu.VMEM((_BPW,), jnp.int32),
        pltpu.VMEM((_BPW, _DE), jnp.float32),
        pltpu.VMEM((_BPW, _DE), jnp.float32),
        pltpu.VMEM((_BPW, _DE), jnp.float32),
        pltpu.VMEM((_BPW, _DE), jnp.float32),
        pltpu.VMEM((_BPW, 2 * _DE), jnp.float32),
        pltpu.VMEM((_BPW,), jnp.float32),
        pltpu.VMEM((_BPW,), jnp.float32),
        pltpu.SemaphoreType.DMA,
        pltpu.SemaphoreType.DMA,
        pltpu.SemaphoreType.DMA,
        pltpu.SemaphoreType.DMA,
    ],
    compiler_params=pltpu.CompilerParams(needs_layout_passes=False),
)(_sc_body)


def kernel(data, entities, relations, mr):
    s = _compute_s(mr, relations)
    ih = data[:, 0]
    it = data[:, 1]
    ir = data[:, 2]
    ich = data[:, 3]
    ict = data[:, 4]
    d_pos, d_neg = _sc_call(ih, it, ir, ich, ict, entities, s)
    t_lab = -jnp.ones((_B, 1), dtype=jnp.float32)
    return (d_pos, d_neg, t_lab)
